# baseline (speedup 1.0000x reference)
import functools
import math

import jax
import jax.numpy as jnp
from jax import lax
from jax.experimental import pallas as pl
from jax.experimental.pallas import tpu as pltpu
from jax.experimental.pallas import tpu_sc as plsc

_VOCAB = 1000000
_EMBED = 64
_SCALE = math.sqrt(_EMBED)

_NUM_CORES = 2
_NUM_SUBCORES = 16
_NW = _NUM_CORES * _NUM_SUBCORES

_LANES = 16
_CHUNK = 512


def _emb_kernel(n_total: int):
    assert n_total % (_NW * _CHUNK) == 0
    b_per_w = n_total // _NW
    n_chunks = b_per_w // _CHUNK
    mesh = plsc.VectorSubcoreMesh(core_axis_name="c", subcore_axis_name="s")

    @functools.partial(
        pl.kernel,
        mesh=mesh,
        out_type=jax.ShapeDtypeStruct((n_total, _EMBED), jnp.float32),
        scratch_types=[
            pltpu.VMEM((_CHUNK,), jnp.int32),
            pltpu.VMEM((_CHUNK, _EMBED), jnp.float32),
            pltpu.SemaphoreType.DMA,
        ],
        compiler_params=pltpu.CompilerParams(use_tc_tiling_on_sc=False),
    )
    def k(idx_hbm, table_hbm, out_hbm, idx_v, rows_v, sem):
        wid = lax.axis_index("s") * _NUM_CORES + lax.axis_index("c")
        base = wid * b_per_w

        def chunk_body(i, carry):
            off = base + i * _CHUNK
            pltpu.sync_copy(idx_hbm.at[pl.ds(off, _CHUNK)], idx_v)
            pltpu.async_copy(table_hbm.at[idx_v], rows_v, sem).wait()

            def scale_body(r, c):
                for j in range(_EMBED // _LANES):
                    sl = pl.ds(j * _LANES, _LANES)
                    rows_v[r, sl] = rows_v[r, sl] * _SCALE
                return c

            lax.fori_loop(0, _CHUNK, scale_body, 0)
            pltpu.sync_copy(rows_v, out_hbm.at[pl.ds(off, _CHUNK)])
            return carry

        lax.fori_loop(0, n_chunks, chunk_body, 0)

    return k


@jax.jit
def kernel(x, table):
    idx = x.reshape(-1).astype(jnp.int32)
    out = _emb_kernel(idx.shape[0])(idx, table)
    return out.reshape(x.shape + (_EMBED,))


# speedup vs baseline: 1.1330x; 1.1330x over previous
import functools
import math

import jax
import jax.numpy as jnp
from jax import lax
from jax.experimental import pallas as pl
from jax.experimental.pallas import tpu as pltpu
from jax.experimental.pallas import tpu_sc as plsc

_EMBED = 64
_SCALE = math.sqrt(_EMBED)

_NUM_CORES = 2
_NUM_SUBCORES = 16
_NW = _NUM_CORES * _NUM_SUBCORES

_LANES = 16
_CHUNK = 256
_NBUF = 4


def _scale_rows(rows):
    @plsc.parallel_loop(0, _CHUNK, step=1, unroll=8)
    def _(r):
        for j in range(_EMBED // _LANES):
            sl = pl.ds(j * _LANES, _LANES)
            rows[r, sl] = rows[r, sl] * _SCALE


def _emb_kernel(n_total: int):
    assert n_total % (_NW * _CHUNK * _NBUF) == 0
    b_per_w = n_total // _NW
    n_chunks = b_per_w // _CHUNK
    n_groups = n_chunks // _NBUF
    mesh = plsc.VectorSubcoreMesh(core_axis_name="c", subcore_axis_name="s")

    @functools.partial(
        pl.kernel,
        mesh=mesh,
        out_type=jax.ShapeDtypeStruct((n_total, _EMBED), jnp.float32),
        scratch_types=(
            [pltpu.VMEM((b_per_w,), jnp.int32)]
            + [pltpu.VMEM((_CHUNK, _EMBED), jnp.float32) for _ in range(_NBUF)]
            + [pltpu.SemaphoreType.DMA for _ in range(2 * _NBUF)]
        ),
        compiler_params=pltpu.CompilerParams(use_tc_tiling_on_sc=False),
    )
    def k(idx_hbm, table_hbm, out_hbm, idx_all, *bufs_and_sems):
        rows = list(bufs_and_sems[:_NBUF])
        gsem = list(bufs_and_sems[_NBUF : 2 * _NBUF])
        ssem = list(bufs_and_sems[2 * _NBUF :])

        wid = lax.axis_index("s") * _NUM_CORES + lax.axis_index("c")
        base = wid * b_per_w

        pltpu.sync_copy(idx_hbm.at[pl.ds(base, b_per_w)], idx_all)

        def start_gather(i, b):
            idx_view = idx_all.at[pl.ds(i * _CHUNK, _CHUNK)]
            pltpu.async_copy(table_hbm.at[idx_view], rows[b], gsem[b])

        def start_store(i, b):
            pltpu.async_copy(
                rows[b], out_hbm.at[pl.ds(base + i * _CHUNK, _CHUNK)], ssem[b]
            )

        def drain_gather(b):
            pltpu.make_async_copy(table_hbm.at[idx_all.at[pl.ds(0, _CHUNK)]],
                                  rows[b], gsem[b]).wait()

        def drain_store(i, b):
            pltpu.make_async_copy(
                rows[b], out_hbm.at[pl.ds(base + i * _CHUNK, _CHUNK)], ssem[b]
            ).wait()

        for b in range(_NBUF):
            start_gather(b, b)

        def group_body(g, carry):
            i0 = g * _NBUF
            for b in range(_NBUF):
                drain_gather(b)
                _scale_rows(rows[b])
                start_store(i0 + b, b)
            for b in range(_NBUF):
                drain_store(i0 + b, b)
                start_gather(i0 + _NBUF + b, b)
            return carry

        lax.fori_loop(0, n_groups - 1, group_body, 0)

        i0 = (n_groups - 1) * _NBUF
        for b in range(_NBUF):
            drain_gather(b)
            _scale_rows(rows[b])
            start_store(i0 + b, b)
        for b in range(_NBUF):
            drain_store(i0 + b, b)

    return k


@jax.jit
def kernel(x, table):
    idx = x.reshape(-1).astype(jnp.int32)
    out = _emb_kernel(idx.shape[0])(idx, table)
    return out.reshape(x.shape + (_EMBED,))


# speedup vs baseline: 1.3838x; 1.2214x over previous
import functools
import math

import jax
import jax.numpy as jnp
from jax import lax
from jax.experimental import pallas as pl
from jax.experimental.pallas import tpu as pltpu
from jax.experimental.pallas import tpu_sc as plsc

_EMBED = 64
_PADDED = 128
_SCALE = math.sqrt(_EMBED)

_NUM_CORES = 2
_NUM_SUBCORES = 16
_NW = _NUM_CORES * _NUM_SUBCORES

_LANES = 16
_CHUNK = 200
_NBUF = 4


def _scale_rows(rows):
    @plsc.parallel_loop(0, _CHUNK, step=1, unroll=8)
    def _(r):
        for j in range(_EMBED // _LANES):
            sl = pl.ds(j * _LANES, _LANES)
            rows[r, sl] = rows[r, sl] * _SCALE


def _emb_kernel(n_total: int):
    assert n_total % (_NW * _CHUNK * _NBUF) == 0
    b_per_w = n_total // _NW
    n_chunks = b_per_w // _CHUNK
    n_groups = n_chunks // _NBUF
    mesh = plsc.VectorSubcoreMesh(core_axis_name="c", subcore_axis_name="s")

    @functools.partial(
        pl.kernel,
        mesh=mesh,
        out_type=jax.ShapeDtypeStruct((n_total, _PADDED), jnp.float32),
        scratch_types=(
            [pltpu.VMEM((_CHUNK,), jnp.int32) for _ in range(_NBUF)]
            + [pltpu.VMEM((_CHUNK, _PADDED), jnp.float32) for _ in range(_NBUF)]
            + [pltpu.SemaphoreType.DMA for _ in range(3 * _NBUF)]
        ),
        compiler_params=pltpu.CompilerParams(use_tc_tiling_on_sc=True),
    )
    def k(idx_hbm, table_hbm, out_hbm, *refs):
        ichunk = list(refs[:_NBUF])
        rows = list(refs[_NBUF : 2 * _NBUF])
        isem = list(refs[2 * _NBUF : 3 * _NBUF])
        gsem = list(refs[3 * _NBUF : 4 * _NBUF])
        ssem = list(refs[4 * _NBUF :])

        wid = lax.axis_index("s") * _NUM_CORES + lax.axis_index("c")
        base = wid * b_per_w

        def start_idx(i, b):
            pltpu.async_copy(
                idx_hbm.at[pl.ds(base + i * _CHUNK, _CHUNK)], ichunk[b], isem[b]
            )

        def start_gather(b):
            pltpu.async_copy(table_hbm.at[ichunk[b]], rows[b], gsem[b])

        def start_store(i, b):
            pltpu.async_copy(
                rows[b],
                out_hbm.at[pl.ds(base + i * _CHUNK, _CHUNK)],
                ssem[b],
            )

        def wait_idx(b):
            pltpu.make_async_copy(
                idx_hbm.at[pl.ds(base, _CHUNK)], ichunk[b], isem[b]
            ).wait()

        def wait_gather(b):
            pltpu.make_async_copy(
                table_hbm.at[ichunk[b]], rows[b], gsem[b]
            ).wait()

        def wait_store(i, b):
            pltpu.make_async_copy(
                rows[b],
                out_hbm.at[pl.ds(base + i * _CHUNK, _CHUNK)],
                ssem[b],
            ).wait()

        for b in range(_NBUF):
            start_idx(b, b)
        for b in range(_NBUF):
            wait_idx(b)
            start_gather(b)

        def group_body(g, carry):
            i0 = g * _NBUF
            for b in range(_NBUF):
                start_idx(i0 + _NBUF + b, b)
            for b in range(_NBUF):
                wait_gather(b)
                _scale_rows(rows[b])
                start_store(i0 + b, b)
            for b in range(_NBUF):
                wait_store(i0 + b, b)
                wait_idx(b)
                start_gather(b)
            return carry

        lax.fori_loop(0, n_groups - 1, group_body, 0)

        i0 = (n_groups - 1) * _NBUF
        for b in range(_NBUF):
            wait_gather(b)
            _scale_rows(rows[b])
            start_store(i0 + b, b)
        for b in range(_NBUF):
            wait_store(i0 + b, b)

    return k


@jax.jit
def kernel(x, table):
    idx = x.reshape(-1).astype(jnp.int32)
    table_padded = jnp.pad(table, ((0, 0), (0, _PADDED - _EMBED)))
    out = _emb_kernel(idx.shape[0])(idx, table_padded)
    return out[:, :_EMBED].reshape(x.shape + (_EMBED,))
